# baseline (speedup 1.0000x reference)
import functools

import jax
import jax.numpy as jnp
from jax import lax
from jax.experimental import pallas as pl
from jax.experimental.pallas import tpu as pltpu
from jax.experimental.pallas import tpu_sc as plsc

_NC = 2
_NS = 16
_NW = _NC * _NS
_L = 16
_K = 90
_CHUNK = 128


def _lfm_call(B):
    b_per_w = B // _NW
    nch = b_per_w // _CHUNK
    mesh = plsc.VectorSubcoreMesh(core_axis_name="c", subcore_axis_name="s")

    @functools.partial(
        pl.kernel,
        mesh=mesh,
        compiler_params=pltpu.CompilerParams(
            needs_layout_passes=False, use_tc_tiling_on_sc=False),
        out_type=jax.ShapeDtypeStruct((_NW, b_per_w), jnp.float32),
        scratch_types=[
            pltpu.VMEM((nch, _CHUNK), jnp.int32),
            pltpu.VMEM((nch, _CHUNK), jnp.int32),
            pltpu.VMEM((nch, _CHUNK, _K), jnp.float32),
            pltpu.VMEM((nch, _CHUNK, _K), jnp.float32),
            pltpu.VMEM((nch, _CHUNK), jnp.float32),
            pltpu.VMEM((nch, _CHUNK), jnp.float32),
            pltpu.VMEM((b_per_w,), jnp.float32),
            pltpu.VMEM((_L, _L + 1), jnp.float32),
            pltpu.SemaphoreType.DMA,
        ],
    )
    def lfm(uidx_hbm, iidx_hbm, p_hbm, q_hbm, bu_hbm, bi_hbm, out_hbm,
            idx_u, idx_i, prow, qrow, buv, biv, outv, tbuf, sem):
        wid = lax.axis_index("s") * _NC + lax.axis_index("c")
        pltpu.sync_copy(uidx_hbm.at[wid], idx_u)
        pltpu.sync_copy(iidx_hbm.at[wid], idx_i)
        copies = []
        for j in range(nch):
            copies.append(pltpu.async_copy(p_hbm.at[idx_u.at[j]], prow.at[j], sem))
            copies.append(pltpu.async_copy(q_hbm.at[idx_i.at[j]], qrow.at[j], sem))
        for c in copies:
            c.wait()

        lanes = lax.iota(jnp.int32, _L)
        tail_keep = lanes >= 6
        zeros = jnp.zeros((_L,), jnp.float32)

        for j in range(nch):
            def body(g, carry, j=j):
                r0 = g * _L
                for i in range(_L):
                    r = r0 + i
                    acc = prow[j, r, pl.ds(0, _L)] * qrow[j, r, pl.ds(0, _L)]
                    for c in range(1, 5):
                        acc = acc + (prow[j, r, pl.ds(c * _L, _L)]
                                     * qrow[j, r, pl.ds(c * _L, _L)])
                    tail = (prow[j, r, pl.ds(_K - _L, _L)]
                            * qrow[j, r, pl.ds(_K - _L, _L)])
                    acc = acc + jnp.where(tail_keep, tail, zeros)
                    tbuf[i, pl.ds(0, _L)] = acc
                dotvec = plsc.load_gather(tbuf, [lanes, jnp.zeros((_L,), jnp.int32)])
                for c in range(1, _L):
                    dotvec = dotvec + plsc.load_gather(
                        tbuf, [lanes, jnp.full((_L,), c, jnp.int32)])
                outv[pl.ds(j * _CHUNK + r0, _L)] = 3.5 + dotvec
                return carry
            lax.fori_loop(0, _CHUNK // _L, body, 0)

        pltpu.sync_copy(outv, out_hbm.at[wid])

    return lfm


def kernel(user_idx, item_idx, P, Q, b_u, b_i):
    B = user_idx.shape[0]
    b_per_w = B // _NW
    nch = b_per_w // _CHUNK
    uidx = user_idx.astype(jnp.int32).reshape(_NW, nch, _CHUNK)
    iidx = item_idx.astype(jnp.int32).reshape(_NW, nch, _CHUNK)
    out = _lfm_call(B)(uidx, iidx, P, Q, b_u, b_i)
    return out.reshape(B)


# speedup vs baseline: 1.0914x; 1.0914x over previous
import functools

import jax
import jax.numpy as jnp
from jax import lax
from jax.experimental import pallas as pl
from jax.experimental.pallas import tpu as pltpu
from jax.experimental.pallas import tpu_sc as plsc

_NC = 2
_NS = 16
_NW = _NC * _NS
_L = 16
_K = 90
_BLK = 128
_WIN = 2 * _BLK
_CHUNK = 128


def _lfm_call(B):
    b_per_w = B // _NW
    nch = b_per_w // _CHUNK
    mesh = plsc.VectorSubcoreMesh(core_axis_name="c", subcore_axis_name="s")

    @functools.partial(
        pl.kernel,
        mesh=mesh,
        compiler_params=pltpu.CompilerParams(
            needs_layout_passes=False, use_tc_tiling_on_sc=False),
        out_type=jax.ShapeDtypeStruct((B,), jnp.float32),
        scratch_types=[
            pltpu.VMEM((2 * b_per_w,), jnp.int32),
            pltpu.VMEM((2 * b_per_w,), jnp.int32),
            pltpu.VMEM((b_per_w,), jnp.int32),
            pltpu.VMEM((b_per_w,), jnp.int32),
            pltpu.VMEM((b_per_w,), jnp.int32),
            pltpu.VMEM((b_per_w,), jnp.int32),
            pltpu.VMEM((b_per_w,), jnp.int32),
            pltpu.VMEM((b_per_w,), jnp.int32),
            pltpu.VMEM((2 * _CHUNK, _BLK), jnp.float32),
            pltpu.VMEM((2 * _CHUNK, _BLK), jnp.float32),
            pltpu.VMEM((_CHUNK, _BLK), jnp.float32),
            pltpu.VMEM((_CHUNK, _BLK), jnp.float32),
            pltpu.VMEM((b_per_w,), jnp.float32),
            pltpu.VMEM((_L, _L + 1), jnp.float32),
            pltpu.SemaphoreType.DMA,
        ],
    )
    def lfm(pblk_hbm, qblk_hbm, poff_hbm, qoff_hbm, bug_hbm, big_hbm,
            buo_hbm, bio_hbm, p_hbm, q_hbm, bu_hbm, bi_hbm, out_hbm,
            pblkv, qblkv, poffv, qoffv, bugv, bigv, buov, biov,
            pgran, qgran, bugran, bigran, outv, tbuf, sem):
        wid = lax.axis_index("s") * _NC + lax.axis_index("c")
        ebase = wid * b_per_w

        pltpu.sync_copy(pblk_hbm.at[pl.ds(2 * ebase, 2 * b_per_w)], pblkv)
        pltpu.sync_copy(qblk_hbm.at[pl.ds(2 * ebase, 2 * b_per_w)], qblkv)
        pltpu.sync_copy(poff_hbm.at[pl.ds(ebase, b_per_w)], poffv)
        pltpu.sync_copy(qoff_hbm.at[pl.ds(ebase, b_per_w)], qoffv)
        pltpu.sync_copy(bug_hbm.at[pl.ds(ebase, b_per_w)], bugv)
        pltpu.sync_copy(big_hbm.at[pl.ds(ebase, b_per_w)], bigv)
        pltpu.sync_copy(buo_hbm.at[pl.ds(ebase, b_per_w)], buov)
        pltpu.sync_copy(bio_hbm.at[pl.ds(ebase, b_per_w)], biov)

        lanes = lax.iota(jnp.int32, _L)
        lanes_win = lanes * _WIN
        tail_keep = lanes >= 6
        zeros = jnp.zeros((_L,), jnp.float32)

        for j in range(nch):
            copies = []
            for h in range(2):
                copies.append(pltpu.async_copy(
                    p_hbm.at[pblkv.at[pl.ds(2 * j * _CHUNK + h * _CHUNK,
                                            _CHUNK)]],
                    pgran.at[pl.ds(h * _CHUNK, _CHUNK)], sem))
                copies.append(pltpu.async_copy(
                    q_hbm.at[qblkv.at[pl.ds(2 * j * _CHUNK + h * _CHUNK,
                                            _CHUNK)]],
                    qgran.at[pl.ds(h * _CHUNK, _CHUNK)], sem))
            copies.append(pltpu.async_copy(
                bu_hbm.at[bugv.at[pl.ds(j * _CHUNK, _CHUNK)]], bugran, sem))
            copies.append(pltpu.async_copy(
                bi_hbm.at[bigv.at[pl.ds(j * _CHUNK, _CHUNK)]], bigran, sem))
            for c in copies:
                c.wait()

            def body(g, carry, j=j):
                e0 = j * _CHUNK + g * _L
                pbase = lanes_win + poffv[pl.ds(e0, _L)]
                qbase = lanes_win + qoffv[pl.ds(e0, _L)]
                gbase = g * _L * _WIN
                for i in range(_L):
                    pflat0 = pbase[i] + gbase
                    qflat0 = qbase[i] + gbase
                    acc = None
                    for c in range(5):
                        fp = pflat0 + (c * _L + lanes)
                        fq = qflat0 + (c * _L + lanes)
                        pv = plsc.load_gather(pgran, [fp >> 7, fp & 127])
                        qv = plsc.load_gather(qgran, [fq >> 7, fq & 127])
                        prod = pv * qv
                        acc = prod if acc is None else acc + prod
                    fp = pflat0 + ((_K - _L) + lanes)
                    fq = qflat0 + ((_K - _L) + lanes)
                    pv = plsc.load_gather(pgran, [fp >> 7, fp & 127])
                    qv = plsc.load_gather(qgran, [fq >> 7, fq & 127])
                    acc = acc + jnp.where(tail_keep, pv * qv, zeros)
                    tbuf[i, pl.ds(0, _L)] = acc
                dotvec = plsc.load_gather(
                    tbuf, [lanes, jnp.zeros((_L,), jnp.int32)])
                for c in range(1, _L):
                    dotvec = dotvec + plsc.load_gather(
                        tbuf, [lanes, jnp.full((_L,), c, jnp.int32)])
                erow = g * _L + lanes
                buval = plsc.load_gather(bugran, [erow, buov[pl.ds(e0, _L)]])
                bival = plsc.load_gather(bigran, [erow, biov[pl.ds(e0, _L)]])
                outv[pl.ds(e0, _L)] = 3.5 + dotvec + buval + bival
                return carry
            lax.fori_loop(0, _CHUNK // _L, body, 0)

        pltpu.sync_copy(outv, out_hbm.at[pl.ds(ebase, b_per_w)])

    return lfm


def _as_blocks(flat):
    n = flat.shape[0]
    npad = (-n) % _BLK
    if npad:
        flat = jnp.concatenate([flat, jnp.zeros((npad,), flat.dtype)])
    return flat.reshape(-1, _BLK)


def kernel(user_idx, item_idx, P, Q, b_u, b_i):
    B = user_idx.shape[0]
    n_users, K = P.shape
    n_items = Q.shape[0]
    ui = user_idx.astype(jnp.int32)
    ii = item_idx.astype(jnp.int32)
    p2 = _as_blocks(P.reshape(-1))
    q2 = _as_blocks(Q.reshape(-1))
    bu2 = _as_blocks(b_u)
    bi2 = _as_blocks(b_i)
    pw = ui * K
    qw = ii * K
    two = jnp.arange(2, dtype=jnp.int32)
    pblk = jnp.minimum((pw >> 7)[:, None] + two[None, :],
                       p2.shape[0] - 1).reshape(-1)
    qblk = jnp.minimum((qw >> 7)[:, None] + two[None, :],
                       q2.shape[0] - 1).reshape(-1)
    out = _lfm_call(B)(
        pblk, qblk, pw & 127, qw & 127, ui >> 7, ii >> 7, ui & 127, ii & 127,
        p2, q2, bu2, bi2)
    return out
